# Initial kernel scaffold; baseline (speedup 1.0000x reference)
import jax
import jax.numpy as jnp
from jax.experimental import pallas as pl


def kernel(input_ids, embed_table):
    raise NotImplementedError("write your pallas kernel here")


# speedup vs baseline: 1.4539x; 1.4539x over previous
import functools

import jax
import jax.numpy as jnp
from jax import lax
from jax.experimental import pallas as pl
from jax.experimental.pallas import tpu as pltpu
from jax.experimental.pallas import tpu_sc as plsc

_info = plsc.get_sparse_core_info()
_NC, _NS = _info.num_cores, _info.num_subcores
_NW = _NC * _NS


def _make_gather(V: int, D: int, N: int):
    b_per_w = N // _NW
    CH = 128
    n_ch = b_per_w // CH
    mesh = plsc.VectorSubcoreMesh(core_axis_name="c", subcore_axis_name="s")

    @functools.partial(
        pl.kernel,
        mesh=mesh,
        out_type=jax.ShapeDtypeStruct((N, D), jnp.float32),
        scratch_types=[
            pltpu.VMEM((b_per_w,), jnp.int32),
            pltpu.VMEM((CH, D), jnp.float32),
            pltpu.SemaphoreType.DMA,
        ],
    )
    def gather(idx_hbm, table_hbm, out_hbm, idx_v, rows_v, sem):
        wid = lax.axis_index("s") * _NC + lax.axis_index("c")
        base = wid * b_per_w
        pltpu.sync_copy(idx_hbm.at[pl.ds(base, b_per_w)], idx_v)
        for c in range(n_ch):
            pltpu.async_copy(
                table_hbm.at[idx_v.at[pl.ds(c * CH, CH)]], rows_v, sem
            ).wait()
            pltpu.sync_copy(rows_v, out_hbm.at[pl.ds(base + c * CH, CH)])

    return gather


def kernel(input_ids, embed_table):
    B, S = input_ids.shape
    V, D = embed_table.shape
    idx = input_ids.reshape(-1).astype(jnp.int32)
    out = _make_gather(V, D, idx.shape[0])(idx, embed_table)
    return out.reshape(B, S, D)


# speedup vs baseline: 1.4602x; 1.0043x over previous
import functools

import jax
import jax.numpy as jnp
from jax import lax
from jax.experimental import pallas as pl
from jax.experimental.pallas import tpu as pltpu
from jax.experimental.pallas import tpu_sc as plsc

_info = plsc.get_sparse_core_info()
_NC, _NS = _info.num_cores, _info.num_subcores
_NW = _NC * _NS


def _make_gather(V: int, D: int, N: int):
    b_per_w = N // _NW
    CH = 64
    n_ch = b_per_w // CH
    mesh = plsc.VectorSubcoreMesh(core_axis_name="c", subcore_axis_name="s")

    @functools.partial(
        pl.kernel,
        mesh=mesh,
        out_type=jax.ShapeDtypeStruct((N, D), jnp.float32),
        scratch_types=[
            pltpu.VMEM((b_per_w,), jnp.int32),
            pltpu.VMEM((2, CH, D), jnp.float32),
            pltpu.SemaphoreType.DMA,
            pltpu.SemaphoreType.DMA,
        ],
    )
    def gather(idx_hbm, table_hbm, out_hbm, idx_v, rows_v, gsem, ssem):
        wid = lax.axis_index("s") * _NC + lax.axis_index("c")
        base = wid * b_per_w
        pltpu.sync_copy(idx_hbm.at[pl.ds(base, b_per_w)], idx_v)
        g = [None] * n_ch
        s = [None] * n_ch
        g[0] = pltpu.async_copy(
            table_hbm.at[idx_v.at[pl.ds(0, CH)]], rows_v.at[0], gsem
        )
        for c in range(n_ch):
            if c + 1 < n_ch:
                if c >= 1:
                    s[c - 1].wait()
                g[c + 1] = pltpu.async_copy(
                    table_hbm.at[idx_v.at[pl.ds((c + 1) * CH, CH)]],
                    rows_v.at[(c + 1) % 2],
                    gsem,
                )
            g[c].wait()
            s[c] = pltpu.async_copy(
                rows_v.at[c % 2], out_hbm.at[pl.ds(base + c * CH, CH)], ssem
            )
        s[n_ch - 2].wait()
        s[n_ch - 1].wait()

    return gather


def kernel(input_ids, embed_table):
    B, S = input_ids.shape
    V, D = embed_table.shape
    idx = input_ids.reshape(-1).astype(jnp.int32)
    out = _make_gather(V, D, idx.shape[0])(idx, embed_table)
    return out.reshape(B, S, D)


# speedup vs baseline: 1.4977x; 1.0257x over previous
import functools

import jax
import jax.numpy as jnp
from jax import lax
from jax.experimental import pallas as pl
from jax.experimental.pallas import tpu as pltpu
from jax.experimental.pallas import tpu_sc as plsc

_info = plsc.get_sparse_core_info()
_NC, _NS = _info.num_cores, _info.num_subcores
_NW = _NC * _NS


def _make_gather(V: int, D: int, N: int):
    b_per_w = N // _NW
    CH = 64
    NBUF = 3
    n_ch = b_per_w // CH
    mesh = plsc.VectorSubcoreMesh(core_axis_name="c", subcore_axis_name="s")

    @functools.partial(
        pl.kernel,
        mesh=mesh,
        out_type=jax.ShapeDtypeStruct((N, D), jnp.float32),
        scratch_types=[
            pltpu.VMEM((b_per_w,), jnp.int32),
            pltpu.VMEM((NBUF, CH, D), jnp.float32),
        ]
        + [pltpu.SemaphoreType.DMA] * (2 * NBUF),
    )
    def gather(idx_hbm, table_hbm, out_hbm, idx_v, rows_v, *sems):
        gsem, ssem = sems[:NBUF], sems[NBUF:]
        wid = lax.axis_index("s") * _NC + lax.axis_index("c")
        base = wid * b_per_w
        pltpu.sync_copy(idx_hbm.at[pl.ds(base, b_per_w)], idx_v)
        g = [None] * n_ch
        s = [None] * n_ch

        def issue_gather(c):
            g[c] = pltpu.async_copy(
                table_hbm.at[idx_v.at[pl.ds(c * CH, CH)]],
                rows_v.at[c % NBUF],
                gsem[c % NBUF],
            )

        for c in range(min(NBUF, n_ch)):
            issue_gather(c)
        for c in range(n_ch):
            g[c].wait()
            s[c] = pltpu.async_copy(
                rows_v.at[c % NBUF],
                out_hbm.at[pl.ds(base + c * CH, CH)],
                ssem[c % NBUF],
            )
            if c + NBUF < n_ch:
                s[c].wait()
                issue_gather(c + NBUF)
        for c in range(max(0, n_ch - NBUF), n_ch):
            s[c].wait()

    return gather


def kernel(input_ids, embed_table):
    B, S = input_ids.shape
    V, D = embed_table.shape
    idx = input_ids.reshape(-1).astype(jnp.int32)
    out = _make_gather(V, D, idx.shape[0])(idx, embed_table)
    return out.reshape(B, S, D)
